# Initial kernel scaffold; baseline (speedup 1.0000x reference)
import jax
import jax.numpy as jnp
from jax.experimental import pallas as pl


def kernel(x, edge_index, edge_weight, W1, b1, W2, b2):
    raise NotImplementedError("write your pallas kernel here")


# speedup vs baseline: 7.2342x; 7.2342x over previous
import functools

import jax
import jax.numpy as jnp
from jax import lax
from jax.experimental import pallas as pl
from jax.experimental.pallas import tpu as pltpu
from jax.experimental.pallas import tpu_sc as plsc

N = 10000
E = 320000
D = 128
NC = 2
NS = 16
L = 16
NW = NC * NS
EPW = 10240
EP = NW * EPW
CH = 128
NCHUNK = EPW // CH
NP = 10240
RPN = NP // NS

_mesh = plsc.VectorSubcoreMesh(core_axis_name="c", subcore_axis_name="s")


@functools.partial(
    pl.kernel,
    mesh=_mesh,
    out_type=jax.ShapeDtypeStruct((NC, NP), jnp.float32),
    scratch_types=[
        pltpu.VMEM((NCHUNK, CH), jnp.int32),
        pltpu.VMEM((NCHUNK, CH), jnp.float32),
        pltpu.VMEM_SHARED((NP,), jnp.float32),
        pltpu.SemaphoreType.DMA,
    ],
)
def _deg_kernel(col_hbm, ew_hbm, zn_hbm, deg_hbm, col_v, ew_v, deg_sh, sem):
    cid = lax.axis_index("c")
    sid = lax.axis_index("s")
    wid = sid * NC + cid
    pltpu.sync_copy(col_hbm.at[wid], col_v)
    pltpu.sync_copy(ew_hbm.at[wid], ew_v)
    pltpu.sync_copy(zn_hbm.at[pl.ds(sid * RPN, RPN)],
                    deg_sh.at[pl.ds(sid * RPN, RPN)])
    plsc.subcore_barrier()

    def chunk(j, carry):
        pltpu.sync_copy(ew_v.at[j], deg_sh.at[col_v.at[j]], add=True)
        return carry

    lax.fori_loop(0, NCHUNK, chunk, 0)
    plsc.subcore_barrier()
    pltpu.sync_copy(deg_sh.at[pl.ds(sid * RPN, RPN)],
                    deg_hbm.at[cid].at[pl.ds(sid * RPN, RPN)])


@functools.partial(
    pl.kernel,
    mesh=_mesh,
    out_type=jax.ShapeDtypeStruct((NC, NP, D), jnp.float32),
    scratch_types=[
        pltpu.VMEM((NCHUNK, CH), jnp.int32),
        pltpu.VMEM((NCHUNK, CH), jnp.int32),
        pltpu.VMEM((NCHUNK, CH), jnp.float32),
        pltpu.VMEM((CH, D), jnp.float32),
        pltpu.VMEM_SHARED((NP, D), jnp.float32),
        pltpu.SemaphoreType.DMA,
    ],
)
def _edge_kernel(row_hbm, col_hbm, ew_hbm, y_hbm, znd_hbm, acc_hbm,
                 row_v, col_v, ew_v, rows_v, acc_sh, sem):
    cid = lax.axis_index("c")
    sid = lax.axis_index("s")
    wid = sid * NC + cid
    pltpu.sync_copy(row_hbm.at[wid], row_v)
    pltpu.sync_copy(col_hbm.at[wid], col_v)
    pltpu.sync_copy(ew_hbm.at[wid], ew_v)
    pltpu.sync_copy(znd_hbm.at[pl.ds(sid * RPN, RPN)],
                    acc_sh.at[pl.ds(sid * RPN, RPN)])
    plsc.subcore_barrier()

    def chunk(j, carry):
        pltpu.async_copy(y_hbm.at[row_v.at[j]], rows_v, sem).wait()

        def grp(g, c2):
            ewv = ew_v[j, pl.ds(g * L, L)]
            for i in range(L):
                s = ewv[i]
                e = g * L + i
                for k in range(D // L):
                    sl = pl.ds(k * L, L)
                    rows_v[e, sl] = rows_v[e, sl] * s
            return c2

        lax.fori_loop(0, CH // L, grp, 0)
        pltpu.sync_copy(rows_v, acc_sh.at[col_v.at[j]], add=True)
        return carry

    lax.fori_loop(0, NCHUNK, chunk, 0)
    plsc.subcore_barrier()
    pltpu.sync_copy(acc_sh.at[pl.ds(sid * RPN, RPN)],
                    acc_hbm.at[cid].at[pl.ds(sid * RPN, RPN)])


def _tc_pre_body(x_ref, w1_ref, deg_ref, y_ref, dis_ref):
    deg = deg_ref[0] + deg_ref[1]
    dis = jnp.where(deg > 0, lax.rsqrt(jnp.where(deg > 0, deg, 1.0)), 0.0)
    dis_ref[...] = dis
    xw = jnp.dot(x_ref[...], w1_ref[...], preferred_element_type=jnp.float32)
    y_ref[...] = xw * dis[:N, None]


def _tc_mid_body(acc_ref, dis_ref, b1_ref, w2_ref, y_ref):
    dis = dis_ref[pl.ds(0, N)]
    acc = acc_ref[0, pl.ds(0, N)] + acc_ref[1, pl.ds(0, N)]
    h = jax.nn.relu(acc * dis[:, None] + b1_ref[...])
    hw = jnp.dot(h, w2_ref[...], preferred_element_type=jnp.float32)
    y_ref[...] = hw * dis[:, None]


def _tc_post_body(acc_ref, dis_ref, b2_ref, out_ref):
    dis = dis_ref[pl.ds(0, N)]
    acc = acc_ref[0, pl.ds(0, N)] + acc_ref[1, pl.ds(0, N)]
    out_ref[...] = jax.nn.relu(acc * dis[:, None] + b2_ref[...])


_tc_pre = pl.pallas_call(
    _tc_pre_body,
    out_shape=(jax.ShapeDtypeStruct((N, D), jnp.float32),
               jax.ShapeDtypeStruct((NP,), jnp.float32)),
)

_tc_mid = pl.pallas_call(
    _tc_mid_body,
    out_shape=jax.ShapeDtypeStruct((N, D), jnp.float32),
)

_tc_post = pl.pallas_call(
    _tc_post_body,
    out_shape=jax.ShapeDtypeStruct((N, D), jnp.float32),
)


def kernel(x, edge_index, edge_weight, W1, b1, W2, b2):
    x = x.astype(jnp.float32)
    row = edge_index[0].astype(jnp.int32)
    col = edge_index[1].astype(jnp.int32)
    ew = edge_weight.astype(jnp.float32)

    pad = EP - E
    row_p = jnp.concatenate([row, jnp.zeros((pad,), jnp.int32)])
    col_p = jnp.concatenate([col, jnp.zeros((pad,), jnp.int32)])
    ew_p = jnp.concatenate([ew, jnp.zeros((pad,), jnp.float32)])
    row_p = row_p.reshape(NW, NCHUNK, CH)
    col_p = col_p.reshape(NW, NCHUNK, CH)
    ew_p = ew_p.reshape(NW, NCHUNK, CH)

    zn = jnp.zeros((NP,), jnp.float32)
    znd = jnp.zeros((NP, D), jnp.float32)

    deg2 = _deg_kernel(col_p, ew_p, zn)
    y1, dis = _tc_pre(x, W1, deg2)
    acc1 = _edge_kernel(row_p, col_p, ew_p, y1, znd)
    y2 = _tc_mid(acc1, dis, b1, W2)
    acc2 = _edge_kernel(row_p, col_p, ew_p, y2, znd)
    return _tc_post(acc2, dis, b2)


# speedup vs baseline: 8.0479x; 1.1125x over previous
import functools

import jax
import jax.numpy as jnp
from jax import lax
from jax.experimental import pallas as pl
from jax.experimental.pallas import tpu as pltpu
from jax.experimental.pallas import tpu_sc as plsc

N = 10000
E = 320000
D = 128
NC = 2
NS = 16
L = 16
NW = NC * NS
EPW = 10240
EP = NW * EPW
CH = 128
NCHUNK = EPW // CH
NP = 10240
RPN = NP // NS

_mesh = plsc.VectorSubcoreMesh(core_axis_name="c", subcore_axis_name="s")


@functools.partial(
    pl.kernel,
    mesh=_mesh,
    out_type=jax.ShapeDtypeStruct((NC, NP), jnp.float32),
    scratch_types=[
        pltpu.VMEM((NCHUNK, CH), jnp.int32),
        pltpu.VMEM((NCHUNK, CH), jnp.float32),
        pltpu.VMEM_SHARED((NP,), jnp.float32),
        pltpu.SemaphoreType.DMA,
    ],
)
def _deg_kernel(col_hbm, ew_hbm, zn_hbm, deg_hbm, col_v, ew_v, deg_sh, sem):
    cid = lax.axis_index("c")
    sid = lax.axis_index("s")
    wid = sid * NC + cid
    pltpu.sync_copy(col_hbm.at[wid], col_v)
    pltpu.sync_copy(ew_hbm.at[wid], ew_v)
    pltpu.sync_copy(zn_hbm.at[pl.ds(sid * RPN, RPN)],
                    deg_sh.at[pl.ds(sid * RPN, RPN)])
    plsc.subcore_barrier()

    def chunk(j, carry):
        pltpu.sync_copy(ew_v.at[j], deg_sh.at[col_v.at[j]], add=True)
        return carry

    lax.fori_loop(0, NCHUNK, chunk, 0)
    plsc.subcore_barrier()
    pltpu.sync_copy(deg_sh.at[pl.ds(sid * RPN, RPN)],
                    deg_hbm.at[cid].at[pl.ds(sid * RPN, RPN)])


@functools.partial(
    pl.kernel,
    mesh=_mesh,
    out_type=jax.ShapeDtypeStruct((NC, NP, D), jnp.float32),
    scratch_types=[
        [pltpu.VMEM((3, CH), jnp.int32) for _ in range(4)],
        [pltpu.VMEM((CH, D), jnp.float32) for _ in range(2)],
        pltpu.VMEM_SHARED((NP, D), jnp.float32),
        [pltpu.SemaphoreType.DMA for _ in range(4)],
        [pltpu.SemaphoreType.DMA for _ in range(2)],
    ],
)
def _edge_kernel(meta_hbm, y_hbm, znd_hbm, acc_hbm,
                 metas, bufs, acc_sh, msems, gsems):
    cid = lax.axis_index("c")
    sid = lax.axis_index("s")
    wid = sid * NC + cid
    mrows = meta_hbm.at[wid]
    pltpu.sync_copy(znd_hbm.at[pl.ds(sid * RPN, RPN)],
                    acc_sh.at[pl.ds(sid * RPN, RPN)])

    def m_start(j, s):
        pltpu.async_copy(mrows.at[j], metas[s], msems[s])

    def m_wait(j, s):
        pltpu.make_async_copy(mrows.at[j], metas[s], msems[s]).wait()

    def g_start(s4, b):
        pltpu.async_copy(y_hbm.at[metas[s4].at[0]], bufs[b], gsems[b])

    def g_wait(s4, b):
        pltpu.make_async_copy(y_hbm.at[metas[s4].at[0]], bufs[b],
                              gsems[b]).wait()

    def a_sync(s4, b):
        pltpu.sync_copy(bufs[b], acc_sh.at[metas[s4].at[1]], add=True)

    def scale(s4, b):
        buf = bufs[b]

        def grp(g, c2):
            ewv = lax.bitcast_convert_type(
                metas[s4][2, pl.ds(g * L, L)], jnp.float32)
            for i in range(L):
                s = ewv[i]
                e = g * L + i
                for k in range(D // L):
                    sl = pl.ds(k * L, L)
                    buf[e, sl] = buf[e, sl] * s
            return c2

        lax.fori_loop(0, CH // L, grp, 0)

    m_start(0, 0)
    m_start(1, 1)
    m_start(2, 2)
    m_wait(0, 0)
    g_start(0, 0)
    plsc.subcore_barrier()

    def quad(jj, carry):
        for u in range(4):
            j = jj * 4 + u
            b2 = u % 2
            g_wait(u, b2)
            if u == 0:
                m_start(j + 3, (u + 3) % 4)
            else:
                @pl.when(jj < NCHUNK // 4 - 1)
                def _m():
                    m_start(j + 3, (u + 3) % 4)
            if u < 3:
                m_wait(j + 1, u + 1)
                g_start(u + 1, 1 - b2)
            else:
                @pl.when(jj < NCHUNK // 4 - 1)
                def _g():
                    m_wait(j + 1, 0)
                    g_start(0, 1 - b2)
            scale(u, b2)
            a_sync(u, b2)
        return carry

    lax.fori_loop(0, NCHUNK // 4, quad, 0)
    plsc.subcore_barrier()
    pltpu.sync_copy(acc_sh.at[pl.ds(sid * RPN, RPN)],
                    acc_hbm.at[cid].at[pl.ds(sid * RPN, RPN)])


def _tc_pre_body(x_ref, w1_ref, deg_ref, y_ref, dis_ref):
    deg = deg_ref[0] + deg_ref[1]
    dis = jnp.where(deg > 0, lax.rsqrt(jnp.where(deg > 0, deg, 1.0)), 0.0)
    dis_ref[...] = dis
    xw = jnp.dot(x_ref[...], w1_ref[...], preferred_element_type=jnp.float32)
    y_ref[...] = xw * dis[:N, None]


def _tc_mid_body(acc_ref, dis_ref, b1_ref, w2_ref, y_ref):
    dis = dis_ref[pl.ds(0, N)]
    acc = acc_ref[0, pl.ds(0, N)] + acc_ref[1, pl.ds(0, N)]
    h = jax.nn.relu(acc * dis[:, None] + b1_ref[...])
    hw = jnp.dot(h, w2_ref[...], preferred_element_type=jnp.float32)
    y_ref[...] = hw * dis[:, None]


def _tc_post_body(acc_ref, dis_ref, b2_ref, out_ref):
    dis = dis_ref[pl.ds(0, N)]
    acc = acc_ref[0, pl.ds(0, N)] + acc_ref[1, pl.ds(0, N)]
    out_ref[...] = jax.nn.relu(acc * dis[:, None] + b2_ref[...])


_tc_pre = pl.pallas_call(
    _tc_pre_body,
    out_shape=(jax.ShapeDtypeStruct((N, D), jnp.float32),
               jax.ShapeDtypeStruct((NP,), jnp.float32)),
)

_tc_mid = pl.pallas_call(
    _tc_mid_body,
    out_shape=jax.ShapeDtypeStruct((N, D), jnp.float32),
)

_tc_post = pl.pallas_call(
    _tc_post_body,
    out_shape=jax.ShapeDtypeStruct((N, D), jnp.float32),
)


def kernel(x, edge_index, edge_weight, W1, b1, W2, b2):
    x = x.astype(jnp.float32)
    row = edge_index[0].astype(jnp.int32)
    col = edge_index[1].astype(jnp.int32)
    ew = edge_weight.astype(jnp.float32)

    pad = EP - E
    row_p = jnp.concatenate([row, jnp.zeros((pad,), jnp.int32)])
    col_p = jnp.concatenate([col, jnp.zeros((pad,), jnp.int32)])
    ew_p = jnp.concatenate([ew, jnp.zeros((pad,), jnp.float32)])
    row_p = row_p.reshape(NW, NCHUNK, CH)
    col_p = col_p.reshape(NW, NCHUNK, CH)
    ew_p = ew_p.reshape(NW, NCHUNK, CH)
    meta = jnp.stack(
        [row_p, col_p, lax.bitcast_convert_type(ew_p, jnp.int32)], axis=2)

    zn = jnp.zeros((NP,), jnp.float32)
    znd = jnp.zeros((NP, D), jnp.float32)

    deg2 = _deg_kernel(col_p, ew_p, zn)
    y1, dis = _tc_pre(x, W1, deg2)
    acc1 = _edge_kernel(meta, y1, znd)
    y2 = _tc_mid(acc1, dis, b1, W2)
    acc2 = _edge_kernel(meta, y2, znd)
    return _tc_post(acc2, dis, b2)


# speedup vs baseline: 8.9182x; 1.1081x over previous
import functools

import jax
import jax.numpy as jnp
from jax import lax
from jax.experimental import pallas as pl
from jax.experimental.pallas import tpu as pltpu
from jax.experimental.pallas import tpu_sc as plsc

N = 10000
E = 320000
D = 128
NC = 2
NS = 16
L = 16
NW = NC * NS
EPW = 10240
EP = NW * EPW
CH = 128
NCHUNK = EPW // CH
NP = 10240
RPN = NP // NS

_mesh = plsc.VectorSubcoreMesh(core_axis_name="c", subcore_axis_name="s")


@functools.partial(
    pl.kernel,
    mesh=_mesh,
    out_type=jax.ShapeDtypeStruct((NC, NP), jnp.float32),
    scratch_types=[
        pltpu.VMEM((NCHUNK, CH), jnp.int32),
        pltpu.VMEM((NCHUNK, CH), jnp.float32),
        pltpu.VMEM_SHARED((NP,), jnp.float32),
        pltpu.SemaphoreType.DMA,
    ],
)
def _deg_kernel(col_hbm, ew_hbm, zn_hbm, deg_hbm, col_v, ew_v, deg_sh, sem):
    cid = lax.axis_index("c")
    sid = lax.axis_index("s")
    wid = sid * NC + cid
    pltpu.sync_copy(col_hbm.at[wid], col_v)
    pltpu.sync_copy(ew_hbm.at[wid], ew_v)
    pltpu.sync_copy(zn_hbm.at[pl.ds(sid * RPN, RPN)],
                    deg_sh.at[pl.ds(sid * RPN, RPN)])
    plsc.subcore_barrier()

    def chunk(j, carry):
        pltpu.sync_copy(ew_v.at[j], deg_sh.at[col_v.at[j]], add=True)
        return carry

    lax.fori_loop(0, NCHUNK, chunk, 0)
    plsc.subcore_barrier()
    pltpu.sync_copy(deg_sh.at[pl.ds(sid * RPN, RPN)],
                    deg_hbm.at[cid].at[pl.ds(sid * RPN, RPN)])


@functools.partial(
    pl.kernel,
    mesh=_mesh,
    out_type=jax.ShapeDtypeStruct((NC, NP, D), jnp.float32),
    scratch_types=[
        [pltpu.VMEM((3, CH), jnp.int32) for _ in range(4)],
        [pltpu.VMEM((CH, D), jnp.float32) for _ in range(2)],
        pltpu.VMEM_SHARED((NP, D), jnp.float32),
        [pltpu.SemaphoreType.DMA for _ in range(4)],
        [pltpu.SemaphoreType.DMA for _ in range(2)],
    ],
)
def _edge_kernel(meta_hbm, y_hbm, znd_hbm, acc_hbm,
                 metas, bufs, acc_sh, msems, gsems):
    cid = lax.axis_index("c")
    sid = lax.axis_index("s")
    wid = sid * NC + cid
    mrows = meta_hbm.at[wid]
    pltpu.sync_copy(znd_hbm.at[pl.ds(sid * RPN, RPN)],
                    acc_sh.at[pl.ds(sid * RPN, RPN)])

    def m_start(j, s):
        pltpu.async_copy(mrows.at[j], metas[s], msems[s])

    def m_wait(j, s):
        pltpu.make_async_copy(mrows.at[j], metas[s], msems[s]).wait()

    yc = y_hbm.at[cid]

    def g_start(s4, b):
        pltpu.async_copy(yc.at[metas[s4].at[0]], bufs[b], gsems[b])

    def g_wait(s4, b):
        pltpu.make_async_copy(yc.at[metas[s4].at[0]], bufs[b],
                              gsems[b]).wait()

    def a_sync(s4, b):
        pltpu.sync_copy(bufs[b], acc_sh.at[metas[s4].at[1]], add=True)

    def scale(s4, b):
        buf = bufs[b]

        def grp(g, c2):
            ewv = lax.bitcast_convert_type(
                metas[s4][2, pl.ds(g * L, L)], jnp.float32)
            for i in range(L):
                s = ewv[i]
                e = g * L + i
                for k in range(D // L):
                    sl = pl.ds(k * L, L)
                    buf[e, sl] = buf[e, sl] * s
            return c2

        lax.fori_loop(0, CH // L, grp, 0)

    m_start(0, 0)
    m_start(1, 1)
    m_start(2, 2)
    m_wait(0, 0)
    g_start(0, 0)
    plsc.subcore_barrier()

    def quad(jj, carry):
        for u in range(4):
            j = jj * 4 + u
            b2 = u % 2
            g_wait(u, b2)
            if u == 0:
                m_start(j + 3, (u + 3) % 4)
            else:
                @pl.when(jj < NCHUNK // 4 - 1)
                def _m():
                    m_start(j + 3, (u + 3) % 4)
            if u < 3:
                m_wait(j + 1, u + 1)
                g_start(u + 1, 1 - b2)
            else:
                @pl.when(jj < NCHUNK // 4 - 1)
                def _g():
                    m_wait(j + 1, 0)
                    g_start(0, 1 - b2)
            scale(u, b2)
            a_sync(u, b2)
        return carry

    lax.fori_loop(0, NCHUNK // 4, quad, 0)
    plsc.subcore_barrier()
    pltpu.sync_copy(acc_sh.at[pl.ds(sid * RPN, RPN)],
                    acc_hbm.at[cid].at[pl.ds(sid * RPN, RPN)])


def _tc_pre_body(x_ref, w1_ref, deg_ref, y_ref, dis_ref):
    deg = deg_ref[0] + deg_ref[1]
    dis = jnp.where(deg > 0, lax.rsqrt(jnp.where(deg > 0, deg, 1.0)), 0.0)
    dis_ref[...] = dis
    xw = jnp.dot(x_ref[...], w1_ref[...], preferred_element_type=jnp.float32)
    v = xw * dis[:N, None]
    y_ref[0] = v
    y_ref[1] = v


def _tc_mid_body(acc_ref, dis_ref, b1_ref, w2_ref, y_ref):
    dis = dis_ref[pl.ds(0, N)]
    acc = acc_ref[0, pl.ds(0, N)] + acc_ref[1, pl.ds(0, N)]
    h = jax.nn.relu(acc * dis[:, None] + b1_ref[...])
    hw = jnp.dot(h, w2_ref[...], preferred_element_type=jnp.float32)
    v = hw * dis[:, None]
    y_ref[0] = v
    y_ref[1] = v


def _tc_post_body(acc_ref, dis_ref, b2_ref, out_ref):
    dis = dis_ref[pl.ds(0, N)]
    acc = acc_ref[0, pl.ds(0, N)] + acc_ref[1, pl.ds(0, N)]
    out_ref[...] = jax.nn.relu(acc * dis[:, None] + b2_ref[...])


_tc_pre = pl.pallas_call(
    _tc_pre_body,
    out_shape=(jax.ShapeDtypeStruct((NC, N, D), jnp.float32),
               jax.ShapeDtypeStruct((NP,), jnp.float32)),
)

_tc_mid = pl.pallas_call(
    _tc_mid_body,
    out_shape=jax.ShapeDtypeStruct((NC, N, D), jnp.float32),
)

_tc_post = pl.pallas_call(
    _tc_post_body,
    out_shape=jax.ShapeDtypeStruct((N, D), jnp.float32),
)


def kernel(x, edge_index, edge_weight, W1, b1, W2, b2):
    x = x.astype(jnp.float32)
    row = edge_index[0].astype(jnp.int32)
    col = edge_index[1].astype(jnp.int32)
    ew = edge_weight.astype(jnp.float32)

    pad = EP - E
    row_p = jnp.concatenate([row, jnp.zeros((pad,), jnp.int32)])
    col_p = jnp.concatenate([col, jnp.zeros((pad,), jnp.int32)])
    ew_p = jnp.concatenate([ew, jnp.zeros((pad,), jnp.float32)])
    row_p = row_p.reshape(NW, NCHUNK, CH)
    col_p = col_p.reshape(NW, NCHUNK, CH)
    ew_p = ew_p.reshape(NW, NCHUNK, CH)
    meta = jnp.stack(
        [row_p, col_p, lax.bitcast_convert_type(ew_p, jnp.int32)], axis=2)

    zn = jnp.zeros((NP,), jnp.float32)
    znd = jnp.zeros((NP, D), jnp.float32)

    deg2 = _deg_kernel(col_p, ew_p, zn)
    y1, dis = _tc_pre(x, W1, deg2)
    acc1 = _edge_kernel(meta, y1, znd)
    y2 = _tc_mid(acc1, dis, b1, W2)
    acc2 = _edge_kernel(meta, y2, znd)
    return _tc_post(acc2, dis, b2)
